# baseline (speedup 1.0000x reference)
import jax
import jax.numpy as jnp
import numpy as np
from jax.experimental import pallas as pl
from jax.experimental.pallas import tpu as pltpu

B = 4; L = 512; C = 7; PRED = 96
D = 128; DFF = 256; E = 8
N = B * C
OUT_W = PRED * C


def _fused_kernel(x_enc_ref, xt_raw_ref, W_in_ref, Wq_ref, Wk_ref, Wv_ref,
                  Wo_ref, Wr_ref, W1_ref, W2_ref, W_out_ref, out_ref,
                  xt_s, hf_s, cw_s, acc_s, sdm_s, mnm_s):
    f32 = jnp.float32
    e_id = pl.program_id(0)

    @pl.when(e_id == 0)
    def _prologue():
        x_enc = x_enc_ref[...]
        xt_raw = xt_raw_ref[...]

        m = jnp.mean(x_enc, axis=1)
        xc3 = x_enc - m[:, None, :]
        m2 = jnp.mean(xc3, axis=1)
        var = jnp.mean((xc3 - m2[:, None, :]) ** 2, axis=1)
        stdev = jnp.sqrt(var + 1e-5)
        rstd = 1.0 / stdev

        n_i = jax.lax.broadcasted_iota(jnp.int32, (N, L), 0)
        j_i = jax.lax.broadcasted_iota(jnp.int32, (N, L), 1)
        cmap = (n_i + j_i) % C
        rn = jax.lax.broadcasted_iota(jnp.int32, (N, B), 0) // C
        rb = jax.lax.broadcasted_iota(jnp.int32, (N, B), 1)
        R = (rn == rb).astype(f32)
        M_n = jnp.dot(R, m, preferred_element_type=f32)
        S_n = jnp.dot(R, rstd, preferred_element_type=f32)
        meanmap = jnp.zeros((N, L), f32)
        rstdmap = jnp.zeros((N, L), f32)
        for c in range(C):
            sel = cmap == c
            meanmap = jnp.where(sel, M_n[:, c][:, None], meanmap)
            rstdmap = jnp.where(sel, S_n[:, c][:, None], rstdmap)
        xt = (xt_raw - meanmap) * rstdmap
        xt_s[...] = xt

        w_in = W_in_ref[...]
        qv = jnp.dot(w_in, Wq_ref[...], preferred_element_type=f32)
        kv = jnp.dot(w_in, Wk_ref[...], preferred_element_type=f32)
        vv = jnp.dot(w_in, Wv_ref[...], preferred_element_type=f32)
        u = jnp.dot(vv, Wo_ref[...], preferred_element_type=f32)
        a = jnp.sum(qv * kv) * (1.0 / np.sqrt(D))

        xl = xt[:, L - 1][:, None]
        logits = (a * xl) * xt
        lmax = jnp.max(logits, axis=1, keepdims=True)
        pexp = jnp.exp(logits - lmax)
        s = (jnp.sum(pexp * xt, axis=1, keepdims=True)
             / jnp.sum(pexp, axis=1, keepdims=True))
        hf = xl * w_in + s * u
        hf_s[...] = hf

        rlog = jnp.dot(hf, Wr_ref[...], preferred_element_type=f32)
        rmax = jnp.max(rlog, axis=1, keepdims=True)
        rexp = jnp.exp(rlog - rmax)
        rp = rexp / jnp.sum(rexp, axis=1, keepdims=True)
        m1 = jnp.max(rp, axis=1, keepdims=True)
        m2v = jnp.max(jnp.where(rp == m1, -1.0, rp), axis=1, keepdims=True)
        cw_s[...] = jnp.where(rp >= m2v, rp, 0.0) / (m1 + m2v)

        ci = jax.lax.broadcasted_iota(jnp.int32, (B, OUT_W), 1) % C
        sdm = jnp.zeros((B, OUT_W), f32)
        mnm = jnp.zeros((B, OUT_W), f32)
        for c in range(C):
            sel = ci == c
            sdm = jnp.where(sel, stdev[:, c][:, None], sdm)
            mnm = jnp.where(sel, m[:, c][:, None], mnm)
        sdm_s[...] = sdm
        mnm_s[...] = mnm

    hf = hf_s[...]
    oh = (jax.lax.broadcasted_iota(jnp.int32, (E, 1), 0) == e_id).astype(f32)
    cwe = jnp.dot(cw_s[...], oh, preferred_element_type=f32)
    g = jnp.dot(hf, W1_ref[0], preferred_element_type=f32)
    ge = g * jax.nn.sigmoid(g)
    ye = jnp.dot(ge, W2_ref[0], preferred_element_type=f32)
    contrib = cwe * ye

    @pl.when(e_id == 0)
    def _init_acc():
        acc_s[...] = contrib

    @pl.when(e_id > 0)
    def _add_acc():
        acc_s[...] = acc_s[...] + contrib

    @pl.when(e_id == E - 1)
    def _epilogue():
        xt = xt_s[...]
        hff = hf_s[...] + acc_s[...]
        preds = jnp.dot(hff, W_out_ref[...], preferred_element_type=f32)

        bi = jax.lax.broadcasted_iota(jnp.int32, (B, N), 0)
        ni = jax.lax.broadcasted_iota(jnp.int32, (B, N), 1)
        S5 = (ni == C * bi + (C - 2)).astype(f32)
        S6 = (ni == C * bi + (C - 1)).astype(f32)
        p5 = jnp.dot(S5, preds, preferred_element_type=f32)
        p6 = jnp.dot(S6, preds, preferred_element_type=f32)
        x6 = jnp.dot(S6, xt, preferred_element_type=f32)
        a_start = L * C - (C - 2) * (L + PRED) - L
        val = jnp.concatenate([p5[:, a_start:], x6, p6], axis=1)
        out_ref[...] = jnp.reshape(val * sdm_s[...] + mnm_s[...],
                                   (B, PRED, C))


def kernel(x_enc, x_mark_enc, x_dec, x_mark_dec, W_in, Wq, Wk, Wv, Wo, Wr,
           W1, W2, W_out):
    xt_raw = jnp.reshape(x_enc, (N, L))
    full = lambda *shape: pl.BlockSpec(shape, lambda e: (0,) * len(shape))
    return pl.pallas_call(
        _fused_kernel,
        grid=(E,),
        in_specs=[
            full(B, L, C),
            full(N, L),
            full(1, D),
            full(D, D),
            full(D, D),
            full(D, D),
            full(D, D),
            full(D, E),
            pl.BlockSpec((1, D, DFF), lambda e: (e, 0, 0)),
            pl.BlockSpec((1, DFF, D), lambda e: (e, 0, 0)),
            full(D, PRED),
        ],
        out_specs=full(B, PRED, C),
        out_shape=jax.ShapeDtypeStruct((B, PRED, C), jnp.float32),
        scratch_shapes=[
            pltpu.VMEM((N, L), jnp.float32),
            pltpu.VMEM((N, D), jnp.float32),
            pltpu.VMEM((N, E), jnp.float32),
            pltpu.VMEM((N, D), jnp.float32),
            pltpu.VMEM((B, OUT_W), jnp.float32),
            pltpu.VMEM((B, OUT_W), jnp.float32),
        ],
        compiler_params=pltpu.CompilerParams(
            dimension_semantics=("arbitrary",),
        ),
    )(x_enc, xt_raw, W_in, Wq, Wk, Wv, Wo, Wr, W1, W2, W_out)


# speedup vs baseline: 1.4213x; 1.4213x over previous
import jax
import jax.numpy as jnp
import numpy as np
from jax.experimental import pallas as pl
from jax.experimental.pallas import tpu as pltpu

B = 4; L = 512; C = 7; PRED = 96
D = 128; DFF = 256; E = 8
N = B * C
OUT_W = PRED * C


def _fused_kernel(x_enc_ref, xt_raw_ref, W_in_ref, Wq_ref, Wk_ref, Wv_ref,
                  Wo_ref, Wr_ref, W1_ref, W2_ref, W_out_ref, out_ref,
                  w1_s, w2_s, sem1, sem2):
    f32 = jnp.float32
    cp1 = pltpu.make_async_copy(W1_ref, w1_s, sem1)
    cp2 = pltpu.make_async_copy(W2_ref, w2_s, sem2)
    cp1.start()
    cp2.start()
    x_enc = x_enc_ref[...]
    xt_raw = xt_raw_ref[...]

    m = jnp.mean(x_enc, axis=1)
    xc = x_enc - m[:, None, :]
    m2 = jnp.mean(xc, axis=1)
    var = jnp.mean((xc - m2[:, None, :]) ** 2, axis=1)
    stdev = jnp.sqrt(var + 1e-5)
    rstd = 1.0 / stdev

    n_i = jax.lax.broadcasted_iota(jnp.int32, (N, L), 0)
    j_i = jax.lax.broadcasted_iota(jnp.int32, (N, L), 1)
    cmap = (n_i + j_i) % C
    rn = jax.lax.broadcasted_iota(jnp.int32, (N, B), 0) // C
    rb = jax.lax.broadcasted_iota(jnp.int32, (N, B), 1)
    R = (rn == rb).astype(f32)
    M_n = jnp.dot(R, m, preferred_element_type=f32)
    S_n = jnp.dot(R, rstd, preferred_element_type=f32)
    meanmap = jnp.zeros((N, L), f32)
    rstdmap = jnp.zeros((N, L), f32)
    for c in range(C):
        sel = cmap == c
        meanmap = jnp.where(sel, M_n[:, c][:, None], meanmap)
        rstdmap = jnp.where(sel, S_n[:, c][:, None], rstdmap)
    xt = (xt_raw - meanmap) * rstdmap

    w_in = W_in_ref[...]
    qv = jnp.dot(w_in, Wq_ref[...], preferred_element_type=f32)
    kv = jnp.dot(w_in, Wk_ref[...], preferred_element_type=f32)
    vv = jnp.dot(w_in, Wv_ref[...], preferred_element_type=f32)
    u = jnp.dot(vv, Wo_ref[...], preferred_element_type=f32)
    a = jnp.sum(qv * kv) * (1.0 / np.sqrt(D))

    xl = xt[:, L - 1][:, None]
    logits = (a * xl) * xt
    lmax = jnp.max(logits, axis=1, keepdims=True)
    pexp = jnp.exp(logits - lmax)
    s = (jnp.sum(pexp * xt, axis=1, keepdims=True)
         / jnp.sum(pexp, axis=1, keepdims=True))
    hf = xl * w_in + s * u

    rlog = jnp.dot(hf, Wr_ref[...], preferred_element_type=f32)
    rmax = jnp.max(rlog, axis=1, keepdims=True)
    rexp = jnp.exp(rlog - rmax)
    rp = rexp / jnp.sum(rexp, axis=1, keepdims=True)
    m1 = jnp.max(rp, axis=1, keepdims=True)
    m2v = jnp.max(jnp.where(rp == m1, -1.0, rp), axis=1, keepdims=True)
    cw = jnp.where(rp >= m2v, rp, 0.0) / (m1 + m2v)

    cp1.wait()
    cp2.wait()
    moe = jnp.zeros((N, D), f32)
    for e in range(E):
        g = jnp.dot(hf, w1_s[e], preferred_element_type=f32)
        ge = g * jax.nn.sigmoid(g)
        ye = jnp.dot(ge, w2_s[e], preferred_element_type=f32)
        moe = moe + cw[:, e][:, None] * ye
    hff = hf + moe
    preds = jnp.dot(hff, W_out_ref[...], preferred_element_type=f32)

    bi = jax.lax.broadcasted_iota(jnp.int32, (B, N), 0)
    ni = jax.lax.broadcasted_iota(jnp.int32, (B, N), 1)
    S5 = (ni == C * bi + (C - 2)).astype(f32)
    S6 = (ni == C * bi + (C - 1)).astype(f32)
    p5 = jnp.dot(S5, preds, preferred_element_type=f32)
    p6 = jnp.dot(S6, preds, preferred_element_type=f32)
    x6 = jnp.dot(S6, xt, preferred_element_type=f32)
    a_start = L * C - (C - 2) * (L + PRED) - L
    val = jnp.concatenate([p5[:, a_start:], x6, p6], axis=1)

    ci = jax.lax.broadcasted_iota(jnp.int32, (B, OUT_W), 1) % C
    sdm = jnp.zeros((B, OUT_W), f32)
    mnm = jnp.zeros((B, OUT_W), f32)
    for c in range(C):
        sel = ci == c
        sdm = jnp.where(sel, stdev[:, c][:, None], sdm)
        mnm = jnp.where(sel, m[:, c][:, None], mnm)
    out_ref[...] = val * sdm + mnm


def kernel(x_enc, x_mark_enc, x_dec, x_mark_dec, W_in, Wq, Wk, Wv, Wo, Wr,
           W1, W2, W_out):
    xt_raw = jnp.reshape(x_enc, (N, L))
    vmem = pl.BlockSpec(memory_space=pltpu.MemorySpace.VMEM)
    hbm = pl.BlockSpec(memory_space=pltpu.MemorySpace.HBM)
    out = pl.pallas_call(
        _fused_kernel,
        in_specs=[vmem, vmem, vmem, vmem, vmem, vmem, vmem, vmem,
                  hbm, hbm, vmem],
        out_specs=vmem,
        out_shape=jax.ShapeDtypeStruct((B, OUT_W), jnp.float32),
        scratch_shapes=[
            pltpu.VMEM((E, D, DFF), jnp.float32),
            pltpu.VMEM((E, DFF, D), jnp.float32),
            pltpu.SemaphoreType.DMA,
            pltpu.SemaphoreType.DMA,
        ],
    )(x_enc, xt_raw, W_in, Wq, Wk, Wv, Wo, Wr, W1, W2, W_out)
    return jnp.reshape(out, (B, PRED, C))
